# baseline (speedup 1.0000x reference)
import functools

import jax
import jax.numpy as jnp
from jax import lax
from jax.experimental import pallas as pl
from jax.experimental.pallas import tpu as pltpu
from jax.experimental.pallas import tpu_sc as plsc

N, H, MH, EH, E = 2048, 1024, 4096, 2048, 8
TB = 256
NT = N // TB
RB = 256
P = 2 * N + E * RB - 2 * RB
NBLK = P // RB
NWORK = 32
TPW = N // NWORK
_SQRT1_2 = 0.7071067811865476


def _gelu(v):
    return 0.5 * v * (1.0 + lax.erf(v * _SQRT1_2))


def _ln(a, g, b, eps=1e-5):
    mu = jnp.mean(a, axis=-1, keepdims=True)
    var = jnp.mean((a - mu) ** 2, axis=-1, keepdims=True)
    return (a - mu) * lax.rsqrt(var + eps) * g + b


def _k1_body(xf_ref, ln1g_ref, ln1b_ref, w1_ref, b1_ref, w2_ref, b2_ref,
             ln2g_ref, ln2b_ref, wr_ref,
             h_ref, gates_ref, sel1_ref, sel2_ref, rmat_ref, c3_ref):
    x = xf_ref[:]
    xn = _ln(x, ln1g_ref[:], ln1b_ref[:])
    h1 = _gelu(lax.dot_general(xn, w1_ref[:], (((1,), (1,)), ((), ())),
                               preferred_element_type=jnp.float32) + b1_ref[:])
    h2 = lax.dot_general(h1, w2_ref[:], (((1,), (1,)), ((), ())),
                         preferred_element_type=jnp.float32) + b2_ref[:] + x
    h = _ln(h2, ln2g_ref[:], ln2b_ref[:]) + h2
    h_ref[:] = h

    logits = lax.dot_general(h, wr_ref[:], (((1,), (1,)), ((), ())),
                             preferred_element_type=jnp.float32)
    z = jnp.exp(logits - jnp.max(logits, axis=1, keepdims=True))
    iota8 = lax.broadcasted_iota(jnp.int32, (TB, E), 1)
    m1 = jnp.max(z, axis=1, keepdims=True)
    i1 = jnp.min(jnp.where(z == m1, iota8, E), axis=1, keepdims=True)
    zm = jnp.where(iota8 == i1, -1.0, z)
    m2 = jnp.max(zm, axis=1, keepdims=True)
    i2 = jnp.min(jnp.where(zm == m2, iota8, E), axis=1, keepdims=True)
    sel1 = (iota8 == i1).astype(jnp.float32)
    sel2 = (iota8 == i2).astype(jnp.float32)
    gates_ref[:] = (sel1 * m1 + sel2 * m2) / (m1 + m2)
    sel1_ref[:] = sel1
    sel2_ref[:] = sel2

    ohb = sel1 + sel2
    r = lax.broadcasted_iota(jnp.int32, (TB, TB), 0)
    c = lax.broadcasted_iota(jnp.int32, (TB, TB), 1)
    strict = (c < r).astype(jnp.float32)
    rmat_ref[:] = lax.dot_general(strict, ohb, (((1,), (0,)), ((), ())),
                                  precision=lax.Precision.HIGHEST,
                              preferred_element_type=jnp.float32)
    c3_ref[:] = jnp.sum(ohb, axis=0).reshape(1, 1, E)


def _k2_body(c3_ref, sel1_ref, sel2_ref, rmat_ref,
             rows_ref, bex_ref, aux_ref):
    c = c3_ref[:].reshape(NT, E)
    crow = jnp.sum(c, axis=0, keepdims=True)
    cp = jnp.ceil(crow * (1.0 / RB)) * RB
    se = lax.broadcasted_iota(jnp.int32, (E, E), 0)
    sc = lax.broadcasted_iota(jnp.int32, (E, E), 1)
    strict_e = (sc < se).astype(jnp.float32)
    pad_off = lax.dot_general(cp, strict_e, (((1,), (1,)), ((), ())),
                              precision=lax.Precision.HIGHEST,
                              preferred_element_type=jnp.float32)

    br = lax.broadcasted_iota(jnp.int32, (NT, NT), 0)
    bc = lax.broadcasted_iota(jnp.int32, (NT, NT), 1)
    strict_b = (bc < br).astype(jnp.float32)
    blockpfx = lax.dot_general(strict_b, c, (((1,), (0,)), ((), ())),
                               precision=lax.Precision.HIGHEST,
                              preferred_element_type=jnp.float32)

    tdiv = lax.shift_right_logical(
        lax.broadcasted_iota(jnp.int32, (N, 1), 0), 8)
    iota_b = lax.broadcasted_iota(jnp.int32, (N, NT), 1)
    bsel = (iota_b == tdiv).astype(jnp.float32)
    bp = lax.dot_general(bsel, blockpfx, (((1,), (0,)), ((), ())),
                         precision=lax.Precision.HIGHEST,
                              preferred_element_type=jnp.float32)
    obe = pad_off + bp + rmat_ref[:]
    row_a = jnp.sum(sel1_ref[:] * obe, axis=1, keepdims=True)
    row_b = jnp.sum(sel2_ref[:] * obe, axis=1, keepdims=True)
    col8 = lax.broadcasted_iota(jnp.int32, (N, E), 1)
    rows_ref[:] = jnp.where(
        col8 == 0, row_a, jnp.where(col8 == 1, row_b, 0.0)).astype(jnp.int32)

    pad_end = pad_off + cp
    j32 = lax.broadcasted_iota(jnp.int32, (NBLK, E), 0).astype(jnp.float32)
    bex = jnp.sum((j32 * RB >= pad_end).astype(jnp.int32), axis=1)
    bex_ref[:] = bex.reshape(1, NBLK)

    load = crow / jnp.sum(crow, axis=1, keepdims=True)
    aux_ref[:] = 0.01 * jnp.sum(load * jnp.log(load + 1e-9), axis=1,
                                keepdims=True)


def _k4_body(bex_ref, xs_ref, w1e_ref, b1e_ref, w2e_ref, b2e_ref, ys_ref):
    j = pl.program_id(0)

    @pl.when(bex_ref[j] < E)
    def _():
        y1 = _gelu(lax.dot_general(xs_ref[:].astype(jnp.bfloat16),
                                   w1e_ref[0].astype(jnp.bfloat16),
                                   (((1,), (1,)), ((), ())),
                                   preferred_element_type=jnp.float32)
                   + b1e_ref[0])
        ys_ref[:] = lax.dot_general(y1.astype(jnp.bfloat16),
                                    w2e_ref[0].astype(jnp.bfloat16),
                                    (((1,), (1,)), ((), ())),
                                    preferred_element_type=jnp.float32) \
            + b2e_ref[0]


def _k5_body(ya_ref, yb_ref, gates_ref, sel1_ref, sel2_ref,
             lnfg_ref, lnfb_ref, out_ref):
    g = gates_ref[:]
    ga = jnp.sum(sel1_ref[:] * g, axis=1, keepdims=True)
    gb = jnp.sum(sel2_ref[:] * g, axis=1, keepdims=True)
    o = ga * ya_ref[:] + gb * yb_ref[:]
    out_ref[:] = _ln(o, lnfg_ref[:], lnfb_ref[:])


def _sc_dispatch_body(h_hbm, idx_hbm, xs_hbm, idxa_v, idxb_v, rows_v, sem):
    wid = lax.axis_index("s") * 2 + lax.axis_index("c")
    base = wid * TPW
    pltpu.sync_copy(h_hbm.at[pl.ds(base, TPW)], rows_v)
    pltpu.sync_copy(idx_hbm.at[wid, 0], idxa_v)
    pltpu.sync_copy(idx_hbm.at[wid, 1], idxb_v)
    cpa = pltpu.async_copy(rows_v, xs_hbm.at[idxa_v], sem)
    cpa.wait()
    cpb = pltpu.async_copy(rows_v, xs_hbm.at[idxb_v], sem)
    cpb.wait()


def _sc_combine_body(ys_hbm, idx_hbm, ya_hbm, yb_hbm, idx_v, rows_v, sem):
    wid = lax.axis_index("s") * 2 + lax.axis_index("c")
    base = wid * TPW
    pltpu.sync_copy(idx_hbm.at[wid, 0], idx_v)
    pltpu.async_copy(ys_hbm.at[idx_v], rows_v, sem).wait()
    pltpu.sync_copy(rows_v, ya_hbm.at[pl.ds(base, TPW)])
    pltpu.sync_copy(idx_hbm.at[wid, 1], idx_v)
    pltpu.async_copy(ys_hbm.at[idx_v], rows_v, sem).wait()
    pltpu.sync_copy(rows_v, yb_hbm.at[pl.ds(base, TPW)])


def kernel(x, ln1_g, ln1_b, w1, b1, w2, b2, ln2_g, ln2_b, wr,
           e_w1, e_b1, e_w2, e_b2, lnf_g, lnf_b):
    xf = x.reshape(N, H)
    row = lambda v: v.reshape(1, -1)

    k1 = pl.pallas_call(
        _k1_body,
        grid=(NT,),
        in_specs=[
            pl.BlockSpec((TB, H), lambda b: (b, 0)),
            pl.BlockSpec((1, H), lambda b: (0, 0)),
            pl.BlockSpec((1, H), lambda b: (0, 0)),
            pl.BlockSpec((MH, H), lambda b: (0, 0)),
            pl.BlockSpec((1, MH), lambda b: (0, 0)),
            pl.BlockSpec((H, MH), lambda b: (0, 0)),
            pl.BlockSpec((1, H), lambda b: (0, 0)),
            pl.BlockSpec((1, H), lambda b: (0, 0)),
            pl.BlockSpec((1, H), lambda b: (0, 0)),
            pl.BlockSpec((E, H), lambda b: (0, 0)),
        ],
        out_specs=[
            pl.BlockSpec((TB, H), lambda b: (b, 0)),
            pl.BlockSpec((TB, E), lambda b: (b, 0)),
            pl.BlockSpec((TB, E), lambda b: (b, 0)),
            pl.BlockSpec((TB, E), lambda b: (b, 0)),
            pl.BlockSpec((TB, E), lambda b: (b, 0)),
            pl.BlockSpec((1, 1, E), lambda b: (b, 0, 0)),
        ],
        out_shape=[
            jax.ShapeDtypeStruct((N, H), jnp.float32),
            jax.ShapeDtypeStruct((N, E), jnp.float32),
            jax.ShapeDtypeStruct((N, E), jnp.float32),
            jax.ShapeDtypeStruct((N, E), jnp.float32),
            jax.ShapeDtypeStruct((N, E), jnp.float32),
            jax.ShapeDtypeStruct((NT, 1, E), jnp.float32),
        ],
    )
    h, gates, sel1, sel2, rmat, c3 = k1(
        xf, row(ln1_g), row(ln1_b), w1, row(b1), w2, row(b2),
        row(ln2_g), row(ln2_b), wr)

    k2 = pl.pallas_call(
        _k2_body,
        grid=(1,),
        in_specs=[
            pl.BlockSpec((NT, 1, E), lambda i: (0, 0, 0)),
            pl.BlockSpec((N, E), lambda i: (0, 0)),
            pl.BlockSpec((N, E), lambda i: (0, 0)),
            pl.BlockSpec((N, E), lambda i: (0, 0)),
        ],
        out_specs=[
            pl.BlockSpec((N, E), lambda i: (0, 0)),
            pl.BlockSpec((1, NBLK), lambda i: (0, 0)),
            pl.BlockSpec((1, 1), lambda i: (0, 0)),
        ],
        out_shape=[
            jax.ShapeDtypeStruct((N, E), jnp.int32),
            jax.ShapeDtypeStruct((1, NBLK), jnp.int32),
            jax.ShapeDtypeStruct((1, 1), jnp.float32),
        ],
    )
    rows, bex, aux = k2(c3, sel1, sel2, rmat)

    idx3 = rows[:, :2].reshape(NWORK, TPW, 2).transpose(0, 2, 1)
    idx3 = jnp.asarray(idx3, jnp.int32)

    mesh = plsc.VectorSubcoreMesh(core_axis_name="c", subcore_axis_name="s")
    sc_dispatch = functools.partial(
        pl.kernel, mesh=mesh,
        out_type=jax.ShapeDtypeStruct((P, H), jnp.float32),
        scratch_types=[
            pltpu.VMEM((TPW,), jnp.int32),
            pltpu.VMEM((TPW,), jnp.int32),
            pltpu.VMEM((TPW, H), jnp.float32),
            pltpu.SemaphoreType.DMA,
        ],
    )(_sc_dispatch_body)
    xs = sc_dispatch(h, idx3)

    k4 = pl.pallas_call(
        _k4_body,
        grid_spec=pltpu.PrefetchScalarGridSpec(
            num_scalar_prefetch=1,
            grid=(NBLK,),
            in_specs=[
                pl.BlockSpec((RB, H), lambda j, bex_s: (j, 0)),
                pl.BlockSpec(
                    (1, EH, H),
                    lambda j, bex_s: (jnp.minimum(bex_s[j], E - 1), 0, 0)),
                pl.BlockSpec(
                    (1, 1, EH),
                    lambda j, bex_s: (jnp.minimum(bex_s[j], E - 1), 0, 0)),
                pl.BlockSpec(
                    (1, H, EH),
                    lambda j, bex_s: (jnp.minimum(bex_s[j], E - 1), 0, 0)),
                pl.BlockSpec(
                    (1, 1, H),
                    lambda j, bex_s: (jnp.minimum(bex_s[j], E - 1), 0, 0)),
            ],
            out_specs=pl.BlockSpec((RB, H), lambda j, bex_s: (j, 0)),
        ),
        out_shape=jax.ShapeDtypeStruct((P, H), jnp.float32),
        compiler_params=pltpu.CompilerParams(
            dimension_semantics=("arbitrary",)),
    )
    ys = k4(bex.reshape(NBLK), xs, e_w1, e_b1.reshape(E, 1, EH), e_w2,
            e_b2.reshape(E, 1, H))

    sc_combine = functools.partial(
        pl.kernel, mesh=mesh,
        out_type=[
            jax.ShapeDtypeStruct((N, H), jnp.float32),
            jax.ShapeDtypeStruct((N, H), jnp.float32),
        ],
        scratch_types=[
            pltpu.VMEM((TPW,), jnp.int32),
            pltpu.VMEM((TPW, H), jnp.float32),
            pltpu.SemaphoreType.DMA,
        ],
    )(_sc_combine_body)
    ya, yb = sc_combine(ys, idx3)

    k5 = pl.pallas_call(
        _k5_body,
        grid=(NT,),
        in_specs=[
            pl.BlockSpec((TB, H), lambda b: (b, 0)),
            pl.BlockSpec((TB, H), lambda b: (b, 0)),
            pl.BlockSpec((TB, E), lambda b: (b, 0)),
            pl.BlockSpec((TB, E), lambda b: (b, 0)),
            pl.BlockSpec((TB, E), lambda b: (b, 0)),
            pl.BlockSpec((1, H), lambda b: (0, 0)),
            pl.BlockSpec((1, H), lambda b: (0, 0)),
        ],
        out_specs=pl.BlockSpec((TB, H), lambda b: (b, 0)),
        out_shape=jax.ShapeDtypeStruct((N, H), jnp.float32),
    )
    outf = k5(ya, yb, gates, sel1, sel2, row(lnf_g), row(lnf_b))

    return outf.reshape(x.shape), aux[0, 0]


# speedup vs baseline: 1.0116x; 1.0116x over previous
import functools

import jax
import jax.numpy as jnp
from jax import lax
from jax.experimental import pallas as pl
from jax.experimental.pallas import tpu as pltpu
from jax.experimental.pallas import tpu_sc as plsc

N, H, MH, EH, E = 2048, 1024, 4096, 2048, 8
TB = 256
NT = N // TB
RB = 256
P = 2 * N + E * RB - 2 * RB
NBLK = P // RB
NWORK = 32
TPW = N // NWORK
_SQRT1_2 = 0.7071067811865476


def _gelu(v):
    return 0.5 * v * (1.0 + lax.erf(v * _SQRT1_2))


def _ln(a, g, b, eps=1e-5):
    mu = jnp.mean(a, axis=-1, keepdims=True)
    var = jnp.mean((a - mu) ** 2, axis=-1, keepdims=True)
    return (a - mu) * lax.rsqrt(var + eps) * g + b


def _k1_body(xf_ref, ln1g_ref, ln1b_ref, w1_ref, b1_ref, w2_ref, b2_ref,
             ln2g_ref, ln2b_ref, wr_ref,
             h_ref, gates_ref, sel1_ref, sel2_ref,
             rows_ref, bex_ref, aux_ref,
             sel1_s, sel2_s, rmat_s, c_s):
    b = pl.program_id(0)
    x = xf_ref[:]
    xn = _ln(x, ln1g_ref[:], ln1b_ref[:])
    h1 = _gelu(lax.dot_general(xn, w1_ref[:], (((1,), (1,)), ((), ())),
                               preferred_element_type=jnp.float32) + b1_ref[:])
    h2 = lax.dot_general(h1, w2_ref[:], (((1,), (1,)), ((), ())),
                         preferred_element_type=jnp.float32) + b2_ref[:] + x
    h = _ln(h2, ln2g_ref[:], ln2b_ref[:]) + h2
    h_ref[:] = h

    logits = lax.dot_general(h, wr_ref[:], (((1,), (1,)), ((), ())),
                             preferred_element_type=jnp.float32)
    z = jnp.exp(logits - jnp.max(logits, axis=1, keepdims=True))
    iota8 = lax.broadcasted_iota(jnp.int32, (TB, E), 1)
    m1 = jnp.max(z, axis=1, keepdims=True)
    i1 = jnp.min(jnp.where(z == m1, iota8, E), axis=1, keepdims=True)
    zm = jnp.where(iota8 == i1, -1.0, z)
    m2 = jnp.max(zm, axis=1, keepdims=True)
    i2 = jnp.min(jnp.where(zm == m2, iota8, E), axis=1, keepdims=True)
    sel1 = (iota8 == i1).astype(jnp.float32)
    sel2 = (iota8 == i2).astype(jnp.float32)
    gates_ref[:] = (sel1 * m1 + sel2 * m2) / (m1 + m2)
    sel1_ref[:] = sel1
    sel2_ref[:] = sel2

    ohb = sel1 + sel2
    r = lax.broadcasted_iota(jnp.int32, (TB, TB), 0)
    c = lax.broadcasted_iota(jnp.int32, (TB, TB), 1)
    strict = (c < r).astype(jnp.float32)
    sl = pl.ds(b * TB, TB)
    sel1_s[sl, :] = sel1
    sel2_s[sl, :] = sel2
    rmat_s[sl, :] = lax.dot_general(strict, ohb, (((1,), (0,)), ((), ())),
                                    precision=lax.Precision.HIGHEST,
                                    preferred_element_type=jnp.float32)
    c_s[pl.ds(b, 1), :] = jnp.sum(ohb, axis=0).reshape(1, E)

    @pl.when(b == NT - 1)
    def _():
        _plan(c_s, sel1_s, sel2_s, rmat_s, rows_ref, bex_ref, aux_ref)


def _plan(c_s, sel1_ref, sel2_ref, rmat_ref,
          rows_ref, bex_ref, aux_ref):
    c = c_s[:]
    crow = jnp.sum(c, axis=0, keepdims=True)
    cp = jnp.ceil(crow * (1.0 / RB)) * RB
    se = lax.broadcasted_iota(jnp.int32, (E, E), 0)
    sc = lax.broadcasted_iota(jnp.int32, (E, E), 1)
    strict_e = (sc < se).astype(jnp.float32)
    pad_off = lax.dot_general(cp, strict_e, (((1,), (1,)), ((), ())),
                              precision=lax.Precision.HIGHEST,
                              preferred_element_type=jnp.float32)

    br = lax.broadcasted_iota(jnp.int32, (NT, NT), 0)
    bc = lax.broadcasted_iota(jnp.int32, (NT, NT), 1)
    strict_b = (bc < br).astype(jnp.float32)
    blockpfx = lax.dot_general(strict_b, c, (((1,), (0,)), ((), ())),
                               precision=lax.Precision.HIGHEST,
                              preferred_element_type=jnp.float32)

    tdiv = lax.shift_right_logical(
        lax.broadcasted_iota(jnp.int32, (N, 1), 0), 8)
    iota_b = lax.broadcasted_iota(jnp.int32, (N, NT), 1)
    bsel = (iota_b == tdiv).astype(jnp.float32)
    bp = lax.dot_general(bsel, blockpfx, (((1,), (0,)), ((), ())),
                         precision=lax.Precision.HIGHEST,
                              preferred_element_type=jnp.float32)
    obe = pad_off + bp + rmat_ref[:]
    row_a = jnp.sum(sel1_ref[:] * obe, axis=1, keepdims=True)
    row_b = jnp.sum(sel2_ref[:] * obe, axis=1, keepdims=True)
    col8 = lax.broadcasted_iota(jnp.int32, (N, E), 1)
    rows_ref[:] = jnp.where(
        col8 == 0, row_a, jnp.where(col8 == 1, row_b, 0.0)).astype(jnp.int32)

    pad_end = pad_off + cp
    j32 = lax.broadcasted_iota(jnp.int32, (NBLK, E), 0).astype(jnp.float32)
    bex = jnp.sum((j32 * RB >= pad_end).astype(jnp.int32), axis=1)
    bex_ref[:] = bex.reshape(1, NBLK)

    load = crow / jnp.sum(crow, axis=1, keepdims=True)
    aux_ref[:] = 0.01 * jnp.sum(load * jnp.log(load + 1e-9), axis=1,
                                keepdims=True)


def _k4_body(bex_ref, xs_ref, w1e_ref, b1e_ref, w2e_ref, b2e_ref, ys_ref):
    j = pl.program_id(0)

    @pl.when(bex_ref[j] < E)
    def _():
        y1 = _gelu(lax.dot_general(xs_ref[:].astype(jnp.bfloat16),
                                   w1e_ref[0].astype(jnp.bfloat16),
                                   (((1,), (1,)), ((), ())),
                                   preferred_element_type=jnp.float32)
                   + b1e_ref[0])
        ys_ref[:] = lax.dot_general(y1.astype(jnp.bfloat16),
                                    w2e_ref[0].astype(jnp.bfloat16),
                                    (((1,), (1,)), ((), ())),
                                    preferred_element_type=jnp.float32) \
            + b2e_ref[0]


def _k5_body(ya_ref, yb_ref, gates_ref, sel1_ref, sel2_ref,
             lnfg_ref, lnfb_ref, out_ref):
    g = gates_ref[:]
    ga = jnp.sum(sel1_ref[:] * g, axis=1, keepdims=True)
    gb = jnp.sum(sel2_ref[:] * g, axis=1, keepdims=True)
    o = ga * ya_ref[:] + gb * yb_ref[:]
    out_ref[:] = _ln(o, lnfg_ref[:], lnfb_ref[:])


def _sc_dispatch_body(h_hbm, idx_hbm, xs_hbm, idxa_v, idxb_v, rows_v, sem):
    wid = lax.axis_index("s") * 2 + lax.axis_index("c")
    base = wid * TPW
    pltpu.sync_copy(h_hbm.at[pl.ds(base, TPW)], rows_v)
    pltpu.sync_copy(idx_hbm.at[wid, 0], idxa_v)
    pltpu.sync_copy(idx_hbm.at[wid, 1], idxb_v)
    cpa = pltpu.async_copy(rows_v, xs_hbm.at[idxa_v], sem)
    cpa.wait()
    cpb = pltpu.async_copy(rows_v, xs_hbm.at[idxb_v], sem)
    cpb.wait()


def _sc_combine_body(ys_hbm, idx_hbm, ya_hbm, yb_hbm, idx_v, rows_v, sem):
    wid = lax.axis_index("s") * 2 + lax.axis_index("c")
    base = wid * TPW
    pltpu.sync_copy(idx_hbm.at[wid, 0], idx_v)
    pltpu.async_copy(ys_hbm.at[idx_v], rows_v, sem).wait()
    pltpu.sync_copy(rows_v, ya_hbm.at[pl.ds(base, TPW)])
    pltpu.sync_copy(idx_hbm.at[wid, 1], idx_v)
    pltpu.async_copy(ys_hbm.at[idx_v], rows_v, sem).wait()
    pltpu.sync_copy(rows_v, yb_hbm.at[pl.ds(base, TPW)])


def kernel(x, ln1_g, ln1_b, w1, b1, w2, b2, ln2_g, ln2_b, wr,
           e_w1, e_b1, e_w2, e_b2, lnf_g, lnf_b):
    xf = x.reshape(N, H)
    row = lambda v: v.reshape(1, -1)

    k1 = pl.pallas_call(
        _k1_body,
        grid=(NT,),
        in_specs=[
            pl.BlockSpec((TB, H), lambda b: (b, 0)),
            pl.BlockSpec((1, H), lambda b: (0, 0)),
            pl.BlockSpec((1, H), lambda b: (0, 0)),
            pl.BlockSpec((MH, H), lambda b: (0, 0)),
            pl.BlockSpec((1, MH), lambda b: (0, 0)),
            pl.BlockSpec((H, MH), lambda b: (0, 0)),
            pl.BlockSpec((1, H), lambda b: (0, 0)),
            pl.BlockSpec((1, H), lambda b: (0, 0)),
            pl.BlockSpec((1, H), lambda b: (0, 0)),
            pl.BlockSpec((E, H), lambda b: (0, 0)),
        ],
        out_specs=[
            pl.BlockSpec((TB, H), lambda b: (b, 0)),
            pl.BlockSpec((TB, E), lambda b: (b, 0)),
            pl.BlockSpec((TB, E), lambda b: (b, 0)),
            pl.BlockSpec((TB, E), lambda b: (b, 0)),
            pl.BlockSpec((N, E), lambda b: (0, 0)),
            pl.BlockSpec((1, NBLK), lambda b: (0, 0)),
            pl.BlockSpec((1, 1), lambda b: (0, 0)),
        ],
        out_shape=[
            jax.ShapeDtypeStruct((N, H), jnp.float32),
            jax.ShapeDtypeStruct((N, E), jnp.float32),
            jax.ShapeDtypeStruct((N, E), jnp.float32),
            jax.ShapeDtypeStruct((N, E), jnp.float32),
            jax.ShapeDtypeStruct((N, E), jnp.int32),
            jax.ShapeDtypeStruct((1, NBLK), jnp.int32),
            jax.ShapeDtypeStruct((1, 1), jnp.float32),
        ],
        scratch_shapes=[
            pltpu.VMEM((N, E), jnp.float32),
            pltpu.VMEM((N, E), jnp.float32),
            pltpu.VMEM((N, E), jnp.float32),
            pltpu.VMEM((NT, E), jnp.float32),
        ],
    )
    h, gates, sel1, sel2, rows, bex, aux = k1(
        xf, row(ln1_g), row(ln1_b), w1, row(b1), w2, row(b2),
        row(ln2_g), row(ln2_b), wr)

    idx3 = rows[:, :2].reshape(NWORK, TPW, 2).transpose(0, 2, 1)
    idx3 = jnp.asarray(idx3, jnp.int32)

    mesh = plsc.VectorSubcoreMesh(core_axis_name="c", subcore_axis_name="s")
    sc_dispatch = functools.partial(
        pl.kernel, mesh=mesh,
        out_type=jax.ShapeDtypeStruct((P, H), jnp.float32),
        scratch_types=[
            pltpu.VMEM((TPW,), jnp.int32),
            pltpu.VMEM((TPW,), jnp.int32),
            pltpu.VMEM((TPW, H), jnp.float32),
            pltpu.SemaphoreType.DMA,
        ],
    )(_sc_dispatch_body)
    xs = sc_dispatch(h, idx3)

    k4 = pl.pallas_call(
        _k4_body,
        grid_spec=pltpu.PrefetchScalarGridSpec(
            num_scalar_prefetch=1,
            grid=(NBLK,),
            in_specs=[
                pl.BlockSpec((RB, H), lambda j, bex_s: (j, 0)),
                pl.BlockSpec(
                    (1, EH, H),
                    lambda j, bex_s: (jnp.minimum(bex_s[j], E - 1), 0, 0)),
                pl.BlockSpec(
                    (1, 1, EH),
                    lambda j, bex_s: (jnp.minimum(bex_s[j], E - 1), 0, 0)),
                pl.BlockSpec(
                    (1, H, EH),
                    lambda j, bex_s: (jnp.minimum(bex_s[j], E - 1), 0, 0)),
                pl.BlockSpec(
                    (1, 1, H),
                    lambda j, bex_s: (jnp.minimum(bex_s[j], E - 1), 0, 0)),
            ],
            out_specs=pl.BlockSpec((RB, H), lambda j, bex_s: (j, 0)),
        ),
        out_shape=jax.ShapeDtypeStruct((P, H), jnp.float32),
        compiler_params=pltpu.CompilerParams(
            dimension_semantics=("arbitrary",)),
    )
    ys = k4(bex.reshape(NBLK), xs, e_w1, e_b1.reshape(E, 1, EH), e_w2,
            e_b2.reshape(E, 1, H))

    sc_combine = functools.partial(
        pl.kernel, mesh=mesh,
        out_type=[
            jax.ShapeDtypeStruct((N, H), jnp.float32),
            jax.ShapeDtypeStruct((N, H), jnp.float32),
        ],
        scratch_types=[
            pltpu.VMEM((TPW,), jnp.int32),
            pltpu.VMEM((TPW, H), jnp.float32),
            pltpu.SemaphoreType.DMA,
        ],
    )(_sc_combine_body)
    ya, yb = sc_combine(ys, idx3)

    k5 = pl.pallas_call(
        _k5_body,
        grid=(NT,),
        in_specs=[
            pl.BlockSpec((TB, H), lambda b: (b, 0)),
            pl.BlockSpec((TB, H), lambda b: (b, 0)),
            pl.BlockSpec((TB, E), lambda b: (b, 0)),
            pl.BlockSpec((TB, E), lambda b: (b, 0)),
            pl.BlockSpec((TB, E), lambda b: (b, 0)),
            pl.BlockSpec((1, H), lambda b: (0, 0)),
            pl.BlockSpec((1, H), lambda b: (0, 0)),
        ],
        out_specs=pl.BlockSpec((TB, H), lambda b: (b, 0)),
        out_shape=jax.ShapeDtypeStruct((N, H), jnp.float32),
    )
    outf = k5(ya, yb, gates, sel1, sel2, row(lnf_g), row(lnf_b))

    return outf.reshape(x.shape), aux[0, 0]
